# baseline (speedup 1.0000x reference)
import jax
import jax.numpy as jnp
import numpy as np
from jax import lax
from jax.experimental import pallas as pl
from jax.experimental.pallas import tpu as pltpu
from jax.experimental.pallas import tpu_sc as plsc

NU = 25001
NI = 25001
N = NU + NI
D = 64
Q = 16
NQ = 4
NB = 3
REG = 0.001

N_PAD = 51200
RPS = N_PAD // 16
WB = 400
C = 1024
WB2 = 200

EPAD_GLOBAL = 2424832
EPAD_BEHAV = 819200

GB_REAL = NB * 4096 * NQ + NB * 4096 * 2 * NQ
GB_PAD = 163840


def _sdt(shape):
    return jax.ShapeDtypeStruct(shape, jnp.float32)


_MESH = dict(
    mesh=plsc.VectorSubcoreMesh(core_axis_name="c", subcore_axis_name="s"),
    compiler_params=pltpu.CompilerParams(use_tc_tiling_on_sc=False),
)

_IDX2 = [pltpu.VMEM((C,), jnp.int32)] * 2
_IDX4 = [pltpu.VMEM((C,), jnp.int32)] * 4
_GB2 = [pltpu.VMEM((C, Q), jnp.float32)] * 2
_SEM = pltpu.SemaphoreType.DMA


def _deg_body(e0, e1, e2, e3, ones128, zeros16, out_hbm,
              idxa, idxb, onesb, zerob, tbuf, acc, sema, semb):
    c = lax.axis_index("c")
    s = lax.axis_index("s")
    w = c * 16 + s
    for r in range(C // 128):
        pltpu.sync_copy(ones128, onesb.at[pl.ds(r * 128, 128)])
    pltpu.sync_copy(zeros16, zerob)
    for g, eref in enumerate((e0, e1, e2, e3)):
        for j in range(RPS // WB):
            pltpu.sync_copy(zerob, acc.at[pl.ds((s * (RPS // WB) + j) * WB, WB)])
        plsc.subcore_barrier()
        epw = eref.shape[0] // 32
        nch = epw // C
        base = w * epw

        def load(i, buf, eref=eref, base=base):
            pltpu.sync_copy(eref.at[pl.ds(base + i * C, C)], buf)

        load(0, idxa)
        M = nch // 2

        def pair(i, _):
            ha = pltpu.async_copy(onesb, acc.at[idxa], sema, add=True)
            load(2 * i + 1, idxb)
            ha.wait()
            hb = pltpu.async_copy(onesb, acc.at[idxb], semb, add=True)
            load(jnp.minimum(2 * i + 2, nch - 1), idxa)
            hb.wait()
            return 0

        lax.fori_loop(0, M, pair, 0)
        if nch % 2 == 1:
            pltpu.async_copy(onesb, acc.at[idxa], sema, add=True).wait()
        plsc.subcore_barrier()
        ob = (g * 2 + c) * N_PAD
        for j in range(RPS // WB):
            r0 = (s * (RPS // WB) + j) * WB
            pltpu.sync_copy(acc.at[pl.ds(r0, WB)], tbuf)
            pltpu.sync_copy(tbuf, out_hbm.at[pl.ds(ob + r0, WB)])
        plsc.subcore_barrier()


def _make_deg_kernel():
    return pl.kernel(
        _deg_body,
        out_type=_sdt((8 * N_PAD, Q)),
        scratch_types=_IDX2 + [
            pltpu.VMEM((C, Q), jnp.float32),
            pltpu.VMEM((WB, Q), jnp.float32),
            pltpu.VMEM((WB, Q), jnp.float32),
            pltpu.VMEM_SHARED((N_PAD, Q), jnp.float32),
            _SEM, _SEM,
        ],
        name="gcn_degrees",
        **_MESH,
    )


def _edge_loop(x_hbm, src2, dst2, acc, off, s,
               sidxa, didxa, sidxb, didxb, gbufa, gbufb,
               gsema, gsemb, ssema, ssemb):
    eps = dst2.shape[0] // 16
    nch = eps // C
    base = s * eps

    def load_idx(i, sidx, didx):
        eb = base + i * C
        pltpu.sync_copy(src2.at[pl.ds(eb, C)], sidx)
        pltpu.sync_copy(dst2.at[pl.ds(eb, C)], didx)
        for k in range(C // 16):
            v = sidx[pl.ds(k * 16, 16)]
            sidx[pl.ds(k * 16, 16)] = v + off

    load_idx(0, sidxa, didxa)
    pltpu.async_copy(x_hbm.at[sidxa], gbufa, gsema)
    M = nch // 2

    def pair(i, _):
        @pl.when(i > 0)
        def _():
            pltpu.make_async_copy(gbufb, acc.at[didxb], ssemb).wait()
        load_idx(2 * i + 1, sidxb, didxb)
        gb = pltpu.async_copy(x_hbm.at[sidxb], gbufb, gsemb)
        pltpu.make_async_copy(x_hbm.at[sidxa], gbufa, gsema).wait()
        sa = pltpu.async_copy(gbufa, acc.at[didxa], ssema, add=True)
        sa.wait()
        load_idx(jnp.minimum(2 * i + 2, nch - 2), sidxa, didxa)
        pltpu.async_copy(x_hbm.at[sidxa], gbufa, gsema)
        gb.wait()
        pltpu.async_copy(gbufb, acc.at[didxb], ssemb, add=True)
        return 0

    lax.fori_loop(0, M, pair, 0)
    pltpu.make_async_copy(x_hbm.at[sidxa], gbufa, gsema).wait()
    pltpu.make_async_copy(gbufb, acc.at[didxb], ssemb).wait()


def _zero_acc(acc, zbuf, s, wb=WB):
    for j in range(RPS // wb):
        pltpu.sync_copy(zbuf, acc.at[pl.ds((s * (RPS // wb) + j) * wb, wb)])


def _prop1_body(x_hbm, src2, dst2, ds_hbm, zeros16, m1_hbm, u1_hbm,
                sidxa, didxa, sidxb, didxb, gbufa, gbufb,
                zbuf, tbuf, tbuf2, dse, acc,
                gsema, gsemb, ssema, ssemb):
    c = lax.axis_index("c")
    s = lax.axis_index("s")
    pltpu.sync_copy(zeros16, zbuf)
    for p in range(2):
        qq = c * 2 + p
        off = qq * N_PAD
        _zero_acc(acc, zbuf, s)
        plsc.subcore_barrier()
        _edge_loop(x_hbm, src2, dst2, acc, off, s,
                   sidxa, didxa, sidxb, didxb, gbufa, gbufb,
                   gsema, gsemb, ssema, ssemb)
        plsc.subcore_barrier()
        for j in range(RPS // WB):
            r0 = (s * (RPS // WB) + j) * WB
            pltpu.sync_copy(acc.at[pl.ds(r0, WB)], tbuf)
            pltpu.sync_copy(ds_hbm.at[pl.ds(r0, WB)], dse)

            def scale(i, _):
                for u in range(8):
                    r = i * 8 + u
                    d = dse[r]
                    m = tbuf[r] * d
                    tbuf[r] = m
                    tbuf2[r] = m * d
                return 0

            lax.fori_loop(0, WB // 8, scale, 0)
            pltpu.sync_copy(tbuf, m1_hbm.at[pl.ds(off + r0, WB)])
            pltpu.sync_copy(tbuf2, u1_hbm.at[pl.ds(off + r0, WB)])
        plsc.subcore_barrier()


def _make_prop1_kernel():
    return pl.kernel(
        _prop1_body,
        out_type=(_sdt((NQ * N_PAD, Q)), _sdt((NQ * N_PAD, Q))),
        scratch_types=_IDX4 + _GB2 + [
            pltpu.VMEM((WB, Q), jnp.float32),
            pltpu.VMEM((WB, Q), jnp.float32),
            pltpu.VMEM((WB, Q), jnp.float32),
            pltpu.VMEM((WB, Q), jnp.float32),
            pltpu.VMEM_SHARED((N_PAD, Q), jnp.float32),
            _SEM, _SEM, _SEM, _SEM,
        ],
        name="gcn_prop1",
        **_MESH,
    )


def _prop2g_body(x_hbm, src2, dst2, ds_hbm, xq_hbm, m1_hbm,
                 db0_hbm, db1_hbm, db2_hbm, zeros16,
                 g_hbm, ub0_hbm, ub1_hbm, ub2_hbm,
                 sidxa, didxa, sidxb, didxb, gbufa, gbufb,
                 zbuf, tbuf, tbuf2, tbuf3, tbuf4, xbuf, mbuf,
                 dse, dbe0, dbe1, dbe2, acc,
                 gsema, gsemb, ssema, ssemb):
    c = lax.axis_index("c")
    s = lax.axis_index("s")
    pltpu.sync_copy(zeros16.at[pl.ds(0, WB2)], zbuf)
    for p in range(2):
        qq = c * 2 + p
        off = qq * N_PAD
        _zero_acc(acc, zbuf, s, WB2)
        plsc.subcore_barrier()
        _edge_loop(x_hbm, src2, dst2, acc, off, s,
                   sidxa, didxa, sidxb, didxb, gbufa, gbufb,
                   gsema, gsemb, ssema, ssemb)
        plsc.subcore_barrier()
        for j in range(RPS // WB2):
            r0 = (s * (RPS // WB2) + j) * WB2
            pltpu.sync_copy(acc.at[pl.ds(r0, WB2)], tbuf)
            pltpu.sync_copy(ds_hbm.at[pl.ds(r0, WB2)], dse)
            pltpu.sync_copy(db0_hbm.at[pl.ds(r0, WB2)], dbe0)
            pltpu.sync_copy(db1_hbm.at[pl.ds(r0, WB2)], dbe1)
            pltpu.sync_copy(db2_hbm.at[pl.ds(r0, WB2)], dbe2)
            pltpu.sync_copy(xq_hbm.at[pl.ds(off + r0, WB2)], xbuf)
            pltpu.sync_copy(m1_hbm.at[pl.ds(off + r0, WB2)], mbuf)

            def scale(i, _):
                for u in range(8):
                    r = i * 8 + u
                    gv = (xbuf[r] + mbuf[r] + tbuf[r] * dse[r]) * (1.0 / 3.0)
                    tbuf[r] = gv
                    tbuf2[r] = gv * dbe0[r]
                    tbuf3[r] = gv * dbe1[r]
                    tbuf4[r] = gv * dbe2[r]
                return 0

            lax.fori_loop(0, WB2 // 8, scale, 0)
            pltpu.sync_copy(tbuf, g_hbm.at[pl.ds(off + r0, WB2)])
            pltpu.sync_copy(tbuf2, ub0_hbm.at[pl.ds(off + r0, WB2)])
            pltpu.sync_copy(tbuf3, ub1_hbm.at[pl.ds(off + r0, WB2)])
            pltpu.sync_copy(tbuf4, ub2_hbm.at[pl.ds(off + r0, WB2)])
        plsc.subcore_barrier()


def _make_prop2g_kernel():
    return pl.kernel(
        _prop2g_body,
        out_type=tuple(_sdt((NQ * N_PAD, Q)) for _ in range(4)),
        scratch_types=_IDX4 + _GB2 + [
            pltpu.VMEM((WB2, Q), jnp.float32),
            pltpu.VMEM((WB2, Q), jnp.float32),
            pltpu.VMEM((WB2, Q), jnp.float32),
            pltpu.VMEM((WB2, Q), jnp.float32),
            pltpu.VMEM((WB2, Q), jnp.float32),
            pltpu.VMEM((WB2, Q), jnp.float32),
            pltpu.VMEM((WB2, Q), jnp.float32),
            pltpu.VMEM((WB2, Q), jnp.float32),
            pltpu.VMEM((WB2, Q), jnp.float32),
            pltpu.VMEM((WB2, Q), jnp.float32),
            pltpu.VMEM((WB2, Q), jnp.float32),
            pltpu.VMEM_SHARED((N_PAD, Q), jnp.float32),
            _SEM, _SEM, _SEM, _SEM,
        ],
        name="gcn_prop2g",
        **_MESH,
    )


def _prop2b_body(x_hbm, src2, dst2, ds_hbm, g_hbm, m1_hbm, zeros16,
                 f_hbm,
                 sidxa, didxa, sidxb, didxb, gbufa, gbufb,
                 zbuf, tbuf, xbuf, mbuf, dse, acc,
                 gsema, gsemb, ssema, ssemb):
    c = lax.axis_index("c")
    s = lax.axis_index("s")
    pltpu.sync_copy(zeros16, zbuf)
    for p in range(2):
        qq = c * 2 + p
        off = qq * N_PAD
        _zero_acc(acc, zbuf, s)
        plsc.subcore_barrier()
        _edge_loop(x_hbm, src2, dst2, acc, off, s,
                   sidxa, didxa, sidxb, didxb, gbufa, gbufb,
                   gsema, gsemb, ssema, ssemb)
        plsc.subcore_barrier()
        for j in range(RPS // WB):
            r0 = (s * (RPS // WB) + j) * WB
            pltpu.sync_copy(acc.at[pl.ds(r0, WB)], tbuf)
            pltpu.sync_copy(ds_hbm.at[pl.ds(r0, WB)], dse)
            pltpu.sync_copy(g_hbm.at[pl.ds(off + r0, WB)], xbuf)
            pltpu.sync_copy(m1_hbm.at[pl.ds(off + r0, WB)], mbuf)

            def scale(i, _):
                for u in range(8):
                    r = i * 8 + u
                    tbuf[r] = (xbuf[r] + mbuf[r] + tbuf[r] * dse[r]) * (1.0 / 3.0)
                return 0

            lax.fori_loop(0, WB // 8, scale, 0)
            pltpu.sync_copy(tbuf, f_hbm.at[pl.ds(off + r0, WB)])
        plsc.subcore_barrier()


def _make_prop2b_kernel():
    return pl.kernel(
        _prop2b_body,
        out_type=_sdt((NQ * N_PAD, Q)),
        scratch_types=_IDX4 + _GB2 + [
            pltpu.VMEM((WB, Q), jnp.float32),
            pltpu.VMEM((WB, Q), jnp.float32),
            pltpu.VMEM((WB, Q), jnp.float32),
            pltpu.VMEM((WB, Q), jnp.float32),
            pltpu.VMEM((WB, Q), jnp.float32),
            pltpu.VMEM_SHARED((N_PAD, Q), jnp.float32),
            _SEM, _SEM, _SEM, _SEM,
        ],
        name="gcn_prop2b",
        **_MESH,
    )


def _bgather_body(f0, f1, f2, gidx, o0, o1, o2,
                  idxa, idxb, buf0, buf1, buf2, sem0, sem1, sem2):
    c = lax.axis_index("c")
    s = lax.axis_index("s")
    w = c * 16 + s
    rows_pw = GB_PAD // 32
    nch = rows_pw // C
    base = w * rows_pw

    def load(i, buf):
        pltpu.sync_copy(gidx.at[pl.ds(base + i * C, C)], buf)

    load(0, idxa)

    def body(i, _):
        h0 = pltpu.async_copy(f0.at[idxa], buf0, sem0)
        h1 = pltpu.async_copy(f1.at[idxa], buf1, sem1)
        h2 = pltpu.async_copy(f2.at[idxa], buf2, sem2)
        load(jnp.minimum(i + 1, nch - 1), idxb)
        h0.wait()
        h1.wait()
        h2.wait()
        ob = base + i * C
        pltpu.sync_copy(buf0, o0.at[pl.ds(ob, C)])
        pltpu.sync_copy(buf1, o1.at[pl.ds(ob, C)])
        pltpu.sync_copy(buf2, o2.at[pl.ds(ob, C)])
        for k in range(C // 16):
            idxa[pl.ds(k * 16, 16)] = idxb[pl.ds(k * 16, 16)]
        return 0

    lax.fori_loop(0, nch, body, 0)


def _make_bgather_kernel():
    return pl.kernel(
        _bgather_body,
        out_type=tuple(_sdt((GB_PAD, Q)) for _ in range(3)),
        scratch_types=_IDX2 + [
            pltpu.VMEM((C, Q), jnp.float32),
            pltpu.VMEM((C, Q), jnp.float32),
            pltpu.VMEM((C, Q), jnp.float32),
            _SEM, _SEM, _SEM,
        ],
        name="gcn_batch_gather",
        **_MESH,
    )


def _build_edges(ei, epad):
    s = ei[0].astype(jnp.int32)
    d = ei[1].astype(jnp.int32) + NU
    src = jnp.concatenate([s, d])
    dst = jnp.concatenate([d, s])
    pad = epad - src.shape[0]
    i = jnp.arange(pad, dtype=jnp.int32)
    psrc = (i * 97) % N
    pdst = N + (i % (N_PAD - N))
    src = jnp.concatenate([src, psrc])
    dst = jnp.concatenate([dst, pdst])
    return src, dst


def _to_quarter(x_pad):
    return x_pad.reshape(N_PAD, NQ, Q).transpose(1, 0, 2).reshape(NQ * N_PAD, Q)


def _mutual_attention(fe, d):
    Bb = fe.shape[1]
    table = []
    feT = jnp.swapaxes(fe, -1, -2)
    for i in range(Bb):
        be = fe[:, i:i + 1, :]
        table.append(jnp.matmul(be, feT))
    last = table[-1]
    norm_num = jnp.sum(last ** 2, axis=1) + 1e-12
    scores = []
    for i in range(Bb - 1):
        res = jnp.sum(last * table[i], axis=1, keepdims=True) * last
        clear = res / norm_num[:, None, :]
        scores.append(clear)
    scores_all = jnp.concatenate(scores, axis=-2)
    s = jnp.sum(jnp.concatenate(scores, axis=-2), axis=-2)[:, None, :] + last
    scores_all = jnp.concatenate([scores_all, s], axis=1)
    att = jax.nn.softmax(scores_all / np.sqrt(d), axis=-1)
    return jnp.matmul(att, fe)


def kernel(user_emb, item_emb, W, item_behaviour_degree, batch_data,
           edge_index_global, edge_index_b0, edge_index_b1, edge_index_b2):
    deg_kernel = _make_deg_kernel()
    prop1 = _make_prop1_kernel()
    prop2g = _make_prop2g_kernel()
    prop2b = _make_prop2b_kernel()
    bgather = _make_bgather_kernel()

    x = jnp.concatenate([user_emb, item_emb], axis=0)
    x_pad = jnp.zeros((N_PAD, D), jnp.float32).at[:N].set(x)
    xq = _to_quarter(x_pad)

    edges = [
        _build_edges(edge_index_global, EPAD_GLOBAL),
        _build_edges(edge_index_b0, EPAD_BEHAV),
        _build_edges(edge_index_b1, EPAD_BEHAV),
        _build_edges(edge_index_b2, EPAD_BEHAV),
    ]

    ones128 = jnp.ones((128, Q), jnp.float32)
    zeros16 = jnp.zeros((WB, Q), jnp.float32)

    degs = deg_kernel(edges[0][1], edges[1][1], edges[2][1], edges[3][1],
                      ones128, zeros16)
    degs = degs.reshape(4, 2, N_PAD, Q)
    dss = []
    for g in range(4):
        deg = degs[g, 0, :, 0] + degs[g, 1, :, 0]
        ds = lax.rsqrt(jnp.where(deg > 0, deg, 1.0))
        dss.append(jnp.broadcast_to(ds[:, None], (N_PAD, Q)))

    u0 = xq * jnp.tile(dss[0][:, 0], NQ)[:, None]
    m1, u1 = prop1(u0, edges[0][0], edges[0][1], dss[0], zeros16)
    g_tab, ub0, ub1, ub2 = prop2g(u1, edges[0][0], edges[0][1], dss[0],
                                  xq, m1, dss[1], dss[2], dss[3], zeros16)

    finals = []
    for k, ub in enumerate((ub0, ub1, ub2)):
        e = edges[k + 1]
        m1b, u1b = prop1(ub, e[0], e[1], dss[k + 1], zeros16)
        fb = prop2b(u1b, e[0], e[1], dss[k + 1], g_tab, m1b, zeros16)
        finals.append(fb)

    bd = batch_data.astype(jnp.int32)
    users = bd[:, :, 0].T
    items = NU + jnp.stack([bd[:, i, 1:3] for i in range(NB)])
    qoff = (jnp.arange(NQ, dtype=jnp.int32) * N_PAD)
    urows = (users[:, :, None] + qoff[None, None, :]).reshape(-1)
    irows = (items[:, :, :, None] + qoff[None, None, None, :]).reshape(-1)
    gidx = jnp.concatenate([urows, irows])
    gidx = jnp.concatenate(
        [gidx, jnp.zeros((GB_PAD - GB_REAL,), jnp.int32)])

    g0, g1, g2 = bgather(finals[0], finals[1], finals[2], gidx)

    nu_rows = NB * 4096 * NQ
    U = [gj[:nu_rows].reshape(NB, 4096, D) for gj in (g0, g1, g2)]
    I = [gj[nu_rows:GB_REAL].reshape(NB, 4096, 2, D) for gj in (g0, g1, g2)]

    weight = item_behaviour_degree * W
    weight = weight / (jnp.sum(weight, axis=1, keepdims=True) + 1e-08)

    total_loss1 = 0.0
    for i in range(NB):
        fe = jnp.stack([U[0][i], U[1][i], U[2][i]], axis=1)
        att = _mutual_attention(fe, D)
        user_feature = att[:, i][:, None, :]
        w_it = weight[bd[:, i, 1:3]]
        item_feature = (I[0][i] * w_it[:, :, 0:1]
                        + I[1][i] * w_it[:, :, 1:2]
                        + I[2][i] * w_it[:, :, 2:3])
        scores = jnp.sum(user_feature * item_feature, axis=2)
        pos, neg = scores[:, 0], scores[:, 1]
        total_loss1 = total_loss1 + (-jnp.mean(jax.nn.log_sigmoid(pos - neg)))
    total_loss = total_loss1 + REG * (
        (jnp.linalg.norm(user_emb) + jnp.linalg.norm(item_emb))
        / item_emb.shape[0])
    return total_loss


# speedup vs baseline: 1.0014x; 1.0014x over previous
import jax
import jax.numpy as jnp
import numpy as np
from jax import lax
from jax.experimental import pallas as pl
from jax.experimental.pallas import tpu as pltpu
from jax.experimental.pallas import tpu_sc as plsc

NU = 25001
NI = 25001
N = NU + NI
D = 64
Q = 16
NQ = 4
NB = 3
REG = 0.001

N_PAD = 51200
RPS = N_PAD // 16
WB = 400
C = 1024
WB2 = 200

EPAD_GLOBAL = 2424832
EPAD_BEHAV = 819200

GB_REAL = NB * 4096 * NQ + NB * 4096 * 2 * NQ
GB_PAD = 163840


def _sdt(shape):
    return jax.ShapeDtypeStruct(shape, jnp.float32)


_MESH = dict(
    mesh=plsc.VectorSubcoreMesh(core_axis_name="c", subcore_axis_name="s"),
    compiler_params=pltpu.CompilerParams(use_tc_tiling_on_sc=False),
)

_IDX2 = [pltpu.VMEM((C,), jnp.int32)] * 2
_IDX4 = [pltpu.VMEM((C,), jnp.int32)] * 4
_GB2 = [pltpu.VMEM((C, Q), jnp.float32)] * 2
_SEM = pltpu.SemaphoreType.DMA


def _deg_body(e0, e1, e2, e3, ones128, zeros16, out_hbm,
              idxa, idxb, onesb, zerob, tbuf, acc, sema, semb):
    c = lax.axis_index("c")
    s = lax.axis_index("s")
    w = c * 16 + s
    for r in range(C // 128):
        pltpu.sync_copy(ones128, onesb.at[pl.ds(r * 128, 128)])
    pltpu.sync_copy(zeros16, zerob)
    for g, eref in enumerate((e0, e1, e2, e3)):
        for j in range(RPS // WB):
            pltpu.sync_copy(zerob, acc.at[pl.ds((s * (RPS // WB) + j) * WB, WB)])
        plsc.subcore_barrier()
        epw = eref.shape[0] // 32
        nch = epw // C
        base = w * epw

        def load(i, buf, eref=eref, base=base):
            pltpu.sync_copy(eref.at[pl.ds(base + i * C, C)], buf)

        load(0, idxa)
        M = nch // 2

        def pair(i, _):
            ha = pltpu.async_copy(onesb, acc.at[idxa], sema, add=True)
            load(2 * i + 1, idxb)
            ha.wait()
            hb = pltpu.async_copy(onesb, acc.at[idxb], semb, add=True)
            load(jnp.minimum(2 * i + 2, nch - 1), idxa)
            hb.wait()
            return 0

        lax.fori_loop(0, M, pair, 0)
        if nch % 2 == 1:
            pltpu.async_copy(onesb, acc.at[idxa], sema, add=True).wait()
        plsc.subcore_barrier()
        ob = (g * 2 + c) * N_PAD
        for j in range(RPS // WB):
            r0 = (s * (RPS // WB) + j) * WB
            pltpu.sync_copy(acc.at[pl.ds(r0, WB)], tbuf)
            pltpu.sync_copy(tbuf, out_hbm.at[pl.ds(ob + r0, WB)])
        plsc.subcore_barrier()


def _make_deg_kernel():
    return pl.kernel(
        _deg_body,
        out_type=_sdt((8 * N_PAD, Q)),
        scratch_types=_IDX2 + [
            pltpu.VMEM((C, Q), jnp.float32),
            pltpu.VMEM((WB, Q), jnp.float32),
            pltpu.VMEM((WB, Q), jnp.float32),
            pltpu.VMEM_SHARED((N_PAD, Q), jnp.float32),
            _SEM, _SEM,
        ],
        name="gcn_degrees",
        **_MESH,
    )


def _edge_loop(x_hbm, src2, dst2, acc, off, s,
               sidxa, didxa, sidxb, didxb, gbufa, gbufb,
               gsema, gsemb, ssema, ssemb):
    eps = dst2.shape[0] // 16
    nch = eps // C
    base = s * eps

    def load_idx(i, sidx, didx):
        eb = base + i * C
        pltpu.sync_copy(src2.at[pl.ds(eb, C)], sidx)
        pltpu.sync_copy(dst2.at[pl.ds(eb, C)], didx)
        for k in range(C // 16):
            v = sidx[pl.ds(k * 16, 16)]
            sidx[pl.ds(k * 16, 16)] = v + off

    load_idx(0, sidxa, didxa)
    pltpu.async_copy(x_hbm.at[sidxa], gbufa, gsema)
    M = nch // 2

    def pair(i, _):
        @pl.when(i > 0)
        def _():
            pltpu.make_async_copy(gbufb, acc.at[didxb], ssemb).wait()
        load_idx(2 * i + 1, sidxb, didxb)
        gb = pltpu.async_copy(x_hbm.at[sidxb], gbufb, gsemb)
        pltpu.make_async_copy(x_hbm.at[sidxa], gbufa, gsema).wait()
        sa = pltpu.async_copy(gbufa, acc.at[didxa], ssema, add=True)
        sa.wait()
        load_idx(jnp.minimum(2 * i + 2, nch - 2), sidxa, didxa)
        pltpu.async_copy(x_hbm.at[sidxa], gbufa, gsema)
        gb.wait()
        pltpu.async_copy(gbufb, acc.at[didxb], ssemb, add=True)
        return 0

    lax.fori_loop(0, M, pair, 0)
    pltpu.make_async_copy(x_hbm.at[sidxa], gbufa, gsema).wait()
    pltpu.make_async_copy(gbufb, acc.at[didxb], ssemb).wait()


def _zero_acc(acc, zbuf, s, wb=WB):
    for j in range(RPS // wb):
        pltpu.sync_copy(zbuf, acc.at[pl.ds((s * (RPS // wb) + j) * wb, wb)])


def _prop1_body(x_hbm, src2, dst2, ds_hbm, zeros16, m1_hbm, u1_hbm,
                sidxa, didxa, sidxb, didxb, gbufa, gbufb,
                zbuf, tbuf, tbuf2, dse, acc,
                gsema, gsemb, ssema, ssemb):
    c = lax.axis_index("c")
    s = lax.axis_index("s")
    pltpu.sync_copy(zeros16, zbuf)
    for p in range(2):
        qq = c * 2 + p
        off = qq * N_PAD
        _zero_acc(acc, zbuf, s)
        plsc.subcore_barrier()
        _edge_loop(x_hbm, src2, dst2, acc, off, s,
                   sidxa, didxa, sidxb, didxb, gbufa, gbufb,
                   gsema, gsemb, ssema, ssemb)
        plsc.subcore_barrier()
        for j in range(RPS // WB):
            r0 = (s * (RPS // WB) + j) * WB
            pltpu.sync_copy(acc.at[pl.ds(r0, WB)], tbuf)
            pltpu.sync_copy(ds_hbm.at[pl.ds(r0, WB)], dse)

            def scale(i, _):
                for u in range(4):
                    r = i * 4 + u
                    d = dse[r]
                    m = tbuf[r] * d
                    tbuf[r] = m
                    tbuf2[r] = m * d
                return 0

            lax.fori_loop(0, WB // 4, scale, 0)
            pltpu.sync_copy(tbuf, m1_hbm.at[pl.ds(off + r0, WB)])
            pltpu.sync_copy(tbuf2, u1_hbm.at[pl.ds(off + r0, WB)])
        plsc.subcore_barrier()


def _make_prop1_kernel():
    return pl.kernel(
        _prop1_body,
        out_type=(_sdt((NQ * N_PAD, Q)), _sdt((NQ * N_PAD, Q))),
        scratch_types=_IDX4 + _GB2 + [
            pltpu.VMEM((WB, Q), jnp.float32),
            pltpu.VMEM((WB, Q), jnp.float32),
            pltpu.VMEM((WB, Q), jnp.float32),
            pltpu.VMEM((WB, Q), jnp.float32),
            pltpu.VMEM_SHARED((N_PAD, Q), jnp.float32),
            _SEM, _SEM, _SEM, _SEM,
        ],
        name="gcn_prop1",
        **_MESH,
    )


def _prop2g_body(x_hbm, src2, dst2, ds_hbm, xq_hbm, m1_hbm,
                 db0_hbm, db1_hbm, db2_hbm, zeros16,
                 g_hbm, ub0_hbm, ub1_hbm, ub2_hbm,
                 sidxa, didxa, sidxb, didxb, gbufa, gbufb,
                 zbuf, tbuf, tbuf2, tbuf3, tbuf4, xbuf, mbuf,
                 dse, dbe0, dbe1, dbe2, acc,
                 gsema, gsemb, ssema, ssemb):
    c = lax.axis_index("c")
    s = lax.axis_index("s")
    pltpu.sync_copy(zeros16.at[pl.ds(0, WB2)], zbuf)
    for p in range(2):
        qq = c * 2 + p
        off = qq * N_PAD
        _zero_acc(acc, zbuf, s, WB2)
        plsc.subcore_barrier()
        _edge_loop(x_hbm, src2, dst2, acc, off, s,
                   sidxa, didxa, sidxb, didxb, gbufa, gbufb,
                   gsema, gsemb, ssema, ssemb)
        plsc.subcore_barrier()
        for j in range(RPS // WB2):
            r0 = (s * (RPS // WB2) + j) * WB2
            pltpu.sync_copy(acc.at[pl.ds(r0, WB2)], tbuf)
            pltpu.sync_copy(ds_hbm.at[pl.ds(r0, WB2)], dse)
            pltpu.sync_copy(db0_hbm.at[pl.ds(r0, WB2)], dbe0)
            pltpu.sync_copy(db1_hbm.at[pl.ds(r0, WB2)], dbe1)
            pltpu.sync_copy(db2_hbm.at[pl.ds(r0, WB2)], dbe2)
            pltpu.sync_copy(xq_hbm.at[pl.ds(off + r0, WB2)], xbuf)
            pltpu.sync_copy(m1_hbm.at[pl.ds(off + r0, WB2)], mbuf)

            def scale(i, _):
                for u in range(4):
                    r = i * 4 + u
                    gv = (xbuf[r] + mbuf[r] + tbuf[r] * dse[r]) * (1.0 / 3.0)
                    tbuf[r] = gv
                    tbuf2[r] = gv * dbe0[r]
                    tbuf3[r] = gv * dbe1[r]
                    tbuf4[r] = gv * dbe2[r]
                return 0

            lax.fori_loop(0, WB2 // 4, scale, 0)
            pltpu.sync_copy(tbuf, g_hbm.at[pl.ds(off + r0, WB2)])
            pltpu.sync_copy(tbuf2, ub0_hbm.at[pl.ds(off + r0, WB2)])
            pltpu.sync_copy(tbuf3, ub1_hbm.at[pl.ds(off + r0, WB2)])
            pltpu.sync_copy(tbuf4, ub2_hbm.at[pl.ds(off + r0, WB2)])
        plsc.subcore_barrier()


def _make_prop2g_kernel():
    return pl.kernel(
        _prop2g_body,
        out_type=tuple(_sdt((NQ * N_PAD, Q)) for _ in range(4)),
        scratch_types=_IDX4 + _GB2 + [
            pltpu.VMEM((WB2, Q), jnp.float32),
            pltpu.VMEM((WB2, Q), jnp.float32),
            pltpu.VMEM((WB2, Q), jnp.float32),
            pltpu.VMEM((WB2, Q), jnp.float32),
            pltpu.VMEM((WB2, Q), jnp.float32),
            pltpu.VMEM((WB2, Q), jnp.float32),
            pltpu.VMEM((WB2, Q), jnp.float32),
            pltpu.VMEM((WB2, Q), jnp.float32),
            pltpu.VMEM((WB2, Q), jnp.float32),
            pltpu.VMEM((WB2, Q), jnp.float32),
            pltpu.VMEM((WB2, Q), jnp.float32),
            pltpu.VMEM_SHARED((N_PAD, Q), jnp.float32),
            _SEM, _SEM, _SEM, _SEM,
        ],
        name="gcn_prop2g",
        **_MESH,
    )


def _prop2b_body(x_hbm, src2, dst2, ds_hbm, g_hbm, m1_hbm, zeros16,
                 f_hbm,
                 sidxa, didxa, sidxb, didxb, gbufa, gbufb,
                 zbuf, tbuf, xbuf, mbuf, dse, acc,
                 gsema, gsemb, ssema, ssemb):
    c = lax.axis_index("c")
    s = lax.axis_index("s")
    pltpu.sync_copy(zeros16, zbuf)
    for p in range(2):
        qq = c * 2 + p
        off = qq * N_PAD
        _zero_acc(acc, zbuf, s)
        plsc.subcore_barrier()
        _edge_loop(x_hbm, src2, dst2, acc, off, s,
                   sidxa, didxa, sidxb, didxb, gbufa, gbufb,
                   gsema, gsemb, ssema, ssemb)
        plsc.subcore_barrier()
        for j in range(RPS // WB):
            r0 = (s * (RPS // WB) + j) * WB
            pltpu.sync_copy(acc.at[pl.ds(r0, WB)], tbuf)
            pltpu.sync_copy(ds_hbm.at[pl.ds(r0, WB)], dse)
            pltpu.sync_copy(g_hbm.at[pl.ds(off + r0, WB)], xbuf)
            pltpu.sync_copy(m1_hbm.at[pl.ds(off + r0, WB)], mbuf)

            def scale(i, _):
                for u in range(4):
                    r = i * 4 + u
                    tbuf[r] = (xbuf[r] + mbuf[r] + tbuf[r] * dse[r]) * (1.0 / 3.0)
                return 0

            lax.fori_loop(0, WB // 4, scale, 0)
            pltpu.sync_copy(tbuf, f_hbm.at[pl.ds(off + r0, WB)])
        plsc.subcore_barrier()


def _make_prop2b_kernel():
    return pl.kernel(
        _prop2b_body,
        out_type=_sdt((NQ * N_PAD, Q)),
        scratch_types=_IDX4 + _GB2 + [
            pltpu.VMEM((WB, Q), jnp.float32),
            pltpu.VMEM((WB, Q), jnp.float32),
            pltpu.VMEM((WB, Q), jnp.float32),
            pltpu.VMEM((WB, Q), jnp.float32),
            pltpu.VMEM((WB, Q), jnp.float32),
            pltpu.VMEM_SHARED((N_PAD, Q), jnp.float32),
            _SEM, _SEM, _SEM, _SEM,
        ],
        name="gcn_prop2b",
        **_MESH,
    )


def _bgather_body(f0, f1, f2, gidx, o0, o1, o2,
                  idxa, idxb, buf0, buf1, buf2, sem0, sem1, sem2):
    c = lax.axis_index("c")
    s = lax.axis_index("s")
    w = c * 16 + s
    rows_pw = GB_PAD // 32
    nch = rows_pw // C
    base = w * rows_pw

    def load(i, buf):
        pltpu.sync_copy(gidx.at[pl.ds(base + i * C, C)], buf)

    load(0, idxa)

    def body(i, _):
        h0 = pltpu.async_copy(f0.at[idxa], buf0, sem0)
        h1 = pltpu.async_copy(f1.at[idxa], buf1, sem1)
        h2 = pltpu.async_copy(f2.at[idxa], buf2, sem2)
        load(jnp.minimum(i + 1, nch - 1), idxb)
        h0.wait()
        h1.wait()
        h2.wait()
        ob = base + i * C
        pltpu.sync_copy(buf0, o0.at[pl.ds(ob, C)])
        pltpu.sync_copy(buf1, o1.at[pl.ds(ob, C)])
        pltpu.sync_copy(buf2, o2.at[pl.ds(ob, C)])
        for k in range(C // 16):
            idxa[pl.ds(k * 16, 16)] = idxb[pl.ds(k * 16, 16)]
        return 0

    lax.fori_loop(0, nch, body, 0)


def _make_bgather_kernel():
    return pl.kernel(
        _bgather_body,
        out_type=tuple(_sdt((GB_PAD, Q)) for _ in range(3)),
        scratch_types=_IDX2 + [
            pltpu.VMEM((C, Q), jnp.float32),
            pltpu.VMEM((C, Q), jnp.float32),
            pltpu.VMEM((C, Q), jnp.float32),
            _SEM, _SEM, _SEM,
        ],
        name="gcn_batch_gather",
        **_MESH,
    )


def _build_edges(ei, epad):
    s = ei[0].astype(jnp.int32)
    d = ei[1].astype(jnp.int32) + NU
    src = jnp.concatenate([s, d])
    dst = jnp.concatenate([d, s])
    pad = epad - src.shape[0]
    i = jnp.arange(pad, dtype=jnp.int32)
    psrc = (i * 97) % N
    pdst = N + (i % (N_PAD - N))
    src = jnp.concatenate([src, psrc])
    dst = jnp.concatenate([dst, pdst])
    return src, dst


def _to_quarter(x_pad):
    return x_pad.reshape(N_PAD, NQ, Q).transpose(1, 0, 2).reshape(NQ * N_PAD, Q)


def _mutual_attention(fe, d):
    Bb = fe.shape[1]
    table = []
    feT = jnp.swapaxes(fe, -1, -2)
    for i in range(Bb):
        be = fe[:, i:i + 1, :]
        table.append(jnp.matmul(be, feT))
    last = table[-1]
    norm_num = jnp.sum(last ** 2, axis=1) + 1e-12
    scores = []
    for i in range(Bb - 1):
        res = jnp.sum(last * table[i], axis=1, keepdims=True) * last
        clear = res / norm_num[:, None, :]
        scores.append(clear)
    scores_all = jnp.concatenate(scores, axis=-2)
    s = jnp.sum(jnp.concatenate(scores, axis=-2), axis=-2)[:, None, :] + last
    scores_all = jnp.concatenate([scores_all, s], axis=1)
    att = jax.nn.softmax(scores_all / np.sqrt(d), axis=-1)
    return jnp.matmul(att, fe)


def kernel(user_emb, item_emb, W, item_behaviour_degree, batch_data,
           edge_index_global, edge_index_b0, edge_index_b1, edge_index_b2):
    deg_kernel = _make_deg_kernel()
    prop1 = _make_prop1_kernel()
    prop2g = _make_prop2g_kernel()
    prop2b = _make_prop2b_kernel()
    bgather = _make_bgather_kernel()

    x = jnp.concatenate([user_emb, item_emb], axis=0)
    x_pad = jnp.zeros((N_PAD, D), jnp.float32).at[:N].set(x)
    xq = _to_quarter(x_pad)

    edges = [
        _build_edges(edge_index_global, EPAD_GLOBAL),
        _build_edges(edge_index_b0, EPAD_BEHAV),
        _build_edges(edge_index_b1, EPAD_BEHAV),
        _build_edges(edge_index_b2, EPAD_BEHAV),
    ]

    ones128 = jnp.ones((128, Q), jnp.float32)
    zeros16 = jnp.zeros((WB, Q), jnp.float32)

    degs = deg_kernel(edges[0][1], edges[1][1], edges[2][1], edges[3][1],
                      ones128, zeros16)
    degs = degs.reshape(4, 2, N_PAD, Q)
    dss = []
    for g in range(4):
        deg = degs[g, 0, :, 0] + degs[g, 1, :, 0]
        ds = lax.rsqrt(jnp.where(deg > 0, deg, 1.0))
        dss.append(jnp.broadcast_to(ds[:, None], (N_PAD, Q)))

    u0 = xq * jnp.tile(dss[0][:, 0], NQ)[:, None]
    m1, u1 = prop1(u0, edges[0][0], edges[0][1], dss[0], zeros16)
    g_tab, ub0, ub1, ub2 = prop2g(u1, edges[0][0], edges[0][1], dss[0],
                                  xq, m1, dss[1], dss[2], dss[3], zeros16)

    finals = []
    for k, ub in enumerate((ub0, ub1, ub2)):
        e = edges[k + 1]
        m1b, u1b = prop1(ub, e[0], e[1], dss[k + 1], zeros16)
        fb = prop2b(u1b, e[0], e[1], dss[k + 1], g_tab, m1b, zeros16)
        finals.append(fb)

    bd = batch_data.astype(jnp.int32)
    users = bd[:, :, 0].T
    items = NU + jnp.stack([bd[:, i, 1:3] for i in range(NB)])
    qoff = (jnp.arange(NQ, dtype=jnp.int32) * N_PAD)
    urows = (users[:, :, None] + qoff[None, None, :]).reshape(-1)
    irows = (items[:, :, :, None] + qoff[None, None, None, :]).reshape(-1)
    gidx = jnp.concatenate([urows, irows])
    gidx = jnp.concatenate(
        [gidx, jnp.zeros((GB_PAD - GB_REAL,), jnp.int32)])

    g0, g1, g2 = bgather(finals[0], finals[1], finals[2], gidx)

    nu_rows = NB * 4096 * NQ
    U = [gj[:nu_rows].reshape(NB, 4096, D) for gj in (g0, g1, g2)]
    I = [gj[nu_rows:GB_REAL].reshape(NB, 4096, 2, D) for gj in (g0, g1, g2)]

    weight = item_behaviour_degree * W
    weight = weight / (jnp.sum(weight, axis=1, keepdims=True) + 1e-08)

    total_loss1 = 0.0
    for i in range(NB):
        fe = jnp.stack([U[0][i], U[1][i], U[2][i]], axis=1)
        att = _mutual_attention(fe, D)
        user_feature = att[:, i][:, None, :]
        w_it = weight[bd[:, i, 1:3]]
        item_feature = (I[0][i] * w_it[:, :, 0:1]
                        + I[1][i] * w_it[:, :, 1:2]
                        + I[2][i] * w_it[:, :, 2:3])
        scores = jnp.sum(user_feature * item_feature, axis=2)
        pos, neg = scores[:, 0], scores[:, 1]
        total_loss1 = total_loss1 + (-jnp.mean(jax.nn.log_sigmoid(pos - neg)))
    total_loss = total_loss1 + REG * (
        (jnp.linalg.norm(user_emb) + jnp.linalg.norm(item_emb))
        / item_emb.shape[0])
    return total_loss
